# baseline (speedup 1.0000x reference)
import jax
import jax.numpy as jnp
from jax import lax
from jax.experimental import pallas as pl
from jax.experimental.pallas import tpu as pltpu
from jax.experimental.pallas import tpu_sc as plsc

BUCKET_W = 4.0
N_ROWS = 1_000_000
D = 64
NW = 32

SC_COLS = 262144
W = 512
NTS = SC_COLS // W
KMAX = NTS // NW

BC = 32768
TC_COLS = N_ROWS - SC_COLS
TC_BLOCK0 = SC_COLS // BC
GRID_T = -(-TC_COLS // BC)

BS = 131072
GRID_B = -(-N_ROWS // BS)
GRID_B_SC = SC_COLS // BS


def _tc_body(x_ref, a_ref, b_ref, hash_ref, min_ref, min_sc):
    i = pl.program_id(0)
    b = b_ref[0]
    proj = jax.lax.dot_general(
        a_ref[...], x_ref[...],
        dimension_numbers=(((1,), (0,)), ((), ())),
        preferred_element_type=jnp.float32,
    )
    h = jnp.floor((proj + b) * (1.0 / BUCKET_W))

    @pl.when(i == 0)
    def _():
        min_sc[0] = jnp.min(h)

    @pl.when((i > 0) & (i < GRID_T - 1))
    def _():
        min_sc[0] = jnp.minimum(min_sc[0], jnp.min(h))

    hash_ref[...] = h.reshape(BC).astype(jnp.int32)

    @pl.when(i == GRID_T - 1)
    def _():
        cols = SC_COLS + i * BC + jax.lax.broadcasted_iota(jnp.int32, (1, BC), 1)
        hm = jnp.min(jnp.where(cols < N_ROWS, h, jnp.inf))
        min_ref[0] = jnp.minimum(min_sc[0], hm).astype(jnp.int32)


def _sc_body(x_hbm, a_hbm, b_hbm, hash_hbm, mins_hbm,
             buf0, buf1, a_v, b_v, outb, mvec, sem0, sem1):
    wid = lax.axis_index("s") * 2 + lax.axis_index("c")
    bufs = (buf0, buf1)
    sems = (sem0, sem1)
    pltpu.sync_copy(a_hbm, a_v)
    pltpu.sync_copy(b_hbm, b_v)
    bvec = b_v[...]
    a_vr = [a_v[pl.ds(g * 16, 16)] for g in range(D // 16)]
    a_s = [a_vr[j // 16][j % 16] for j in range(D)]
    mvec[...] = jnp.full((16,), 1 << 30, jnp.int32)

    def start(k):
        t = wid + k * NW
        pltpu.make_async_copy(
            x_hbm.at[:, pl.ds(t * W, W)], bufs[k % 2], sems[k % 2]
        ).start()

    def compute(k):
        buf = bufs[k % 2]
        pltpu.make_async_copy(
            x_hbm.at[:, pl.ds(0, W)], buf, sems[k % 2]
        ).wait()

        def grp(c, carry):
            col = c * 16
            acc = buf[0, pl.ds(col, 16)] * a_s[0]
            for j in range(1, D):
                acc = acc + buf[j, pl.ds(col, 16)] * a_s[j]
            y = (acc + bvec) * (1.0 / BUCKET_W)
            ti = y.astype(jnp.int32)
            ti = jnp.where(ti.astype(jnp.float32) > y, ti - 1, ti)
            mvec[...] = jnp.minimum(mvec[...], ti)
            outb[pl.ds(col, 16)] = ti
            return carry

        lax.fori_loop(0, W // 16, grp, 0)
        t = wid + k * NW
        pltpu.sync_copy(outb, hash_hbm.at[pl.ds(t * W, W)])

    start(0)
    for k in range(KMAX):
        if k + 1 < KMAX:
            start(k + 1)
        compute(k)

    pltpu.sync_copy(mvec, mins_hbm.at[wid])


def _sub_body(hsc_ref, htc_ref, mtc_ref, msc_ref, o_ref, gm):
    i = pl.program_id(0)

    @pl.when(i == 0)
    def _():
        gm[0] = jnp.minimum(mtc_ref[0], jnp.min(msc_ref[...]))

    m = gm[0]

    @pl.when(i < GRID_B_SC)
    def _():
        o_ref[...] = hsc_ref[...] - m

    @pl.when(i >= GRID_B_SC)
    def _():
        o_ref[...] = htc_ref[...] - m


def kernel(inputs, a, b):
    xt = inputs.T
    a2 = a.reshape(1, D)
    a64 = a.reshape(D)
    b16 = jnp.tile(b, 16)

    mesh = plsc.VectorSubcoreMesh(core_axis_name="c", subcore_axis_name="s")
    hash_sc, mins_sc = pl.kernel(
        _sc_body,
        mesh=mesh,
        out_type=[
            jax.ShapeDtypeStruct((SC_COLS,), jnp.int32),
            jax.ShapeDtypeStruct((NW, 16), jnp.int32),
        ],
        scratch_types=[
            pltpu.VMEM((D, W), jnp.float32),
            pltpu.VMEM((D, W), jnp.float32),
            pltpu.VMEM((D,), jnp.float32),
            pltpu.VMEM((16,), jnp.float32),
            pltpu.VMEM((W,), jnp.int32),
            pltpu.VMEM((16,), jnp.int32),
            pltpu.SemaphoreType.DMA,
            pltpu.SemaphoreType.DMA,
        ],
    )(xt, a64, b16)

    hash_tc, min_tc = pl.pallas_call(
        _tc_body,
        grid=(GRID_T,),
        in_specs=[
            pl.BlockSpec((D, BC), lambda i: (0, TC_BLOCK0 + i)),
            pl.BlockSpec((1, D), lambda i: (0, 0)),
            pl.BlockSpec(memory_space=pltpu.SMEM),
        ],
        out_specs=[
            pl.BlockSpec((BC,), lambda i: (i,)),
            pl.BlockSpec(memory_space=pltpu.SMEM),
        ],
        out_shape=[
            jax.ShapeDtypeStruct((TC_COLS,), jnp.int32),
            jax.ShapeDtypeStruct((1,), jnp.int32),
        ],
        scratch_shapes=[pltpu.SMEM((1,), jnp.float32)],
    )(xt, a2, b)

    out = pl.pallas_call(
        _sub_body,
        grid=(GRID_B,),
        in_specs=[
            pl.BlockSpec((BS,), lambda i: (jnp.minimum(i, GRID_B_SC - 1),)),
            pl.BlockSpec((BS,), lambda i: (jnp.maximum(i - GRID_B_SC, 0),)),
            pl.BlockSpec(memory_space=pltpu.SMEM),
            pl.BlockSpec((NW, 16), lambda i: (0, 0)),
        ],
        out_specs=pl.BlockSpec((BS,), lambda i: (i,)),
        out_shape=jax.ShapeDtypeStruct((N_ROWS,), jnp.int32),
        scratch_shapes=[pltpu.SMEM((1,), jnp.int32)],
    )(hash_sc, hash_tc, min_tc, mins_sc)
    return out


# speedup vs baseline: 1.2781x; 1.2781x over previous
import jax
import jax.numpy as jnp
from jax.experimental import pallas as pl
from jax.experimental.pallas import tpu as pltpu

BUCKET_W = 4.0
N_ROWS = 1_000_000
D = 64
BC = 32768
GRID_A = -(-N_ROWS // BC)
BS = 131072
GRID_B = -(-N_ROWS // BS)


def _proj_body(x_ref, a_ref, b_ref, hash_ref, min_ref, min_sc):
    i = pl.program_id(0)
    b = b_ref[0]
    proj = jax.lax.dot_general(
        a_ref[...], x_ref[...],
        dimension_numbers=(((1,), (0,)), ((), ())),
        preferred_element_type=jnp.float32,
    )
    h = jnp.floor((proj + b) * (1.0 / BUCKET_W))
    cols = i * BC + jax.lax.broadcasted_iota(jnp.int32, (1, BC), 1)
    hmin = jnp.min(jnp.where(cols < N_ROWS, h, jnp.inf))

    @pl.when(i == 0)
    def _():
        min_sc[0] = hmin

    @pl.when(i > 0)
    def _():
        min_sc[0] = jnp.minimum(min_sc[0], hmin)

    hash_ref[...] = h.reshape(BC).astype(jnp.int32)

    @pl.when(i == GRID_A - 1)
    def _():
        min_ref[0] = min_sc[0].astype(jnp.int32)


def _sub_body(h_ref, m_ref, o_ref):
    o_ref[...] = h_ref[...] - m_ref[0]


def kernel(inputs, a, b):
    xt = inputs.T
    a2 = a.reshape(1, D)
    hash_u, minv = pl.pallas_call(
        _proj_body,
        grid=(GRID_A,),
        in_specs=[
            pl.BlockSpec((D, BC), lambda i: (0, i)),
            pl.BlockSpec((1, D), lambda i: (0, 0)),
            pl.BlockSpec(memory_space=pltpu.SMEM),
        ],
        out_specs=[
            pl.BlockSpec((BC,), lambda i: (i,)),
            pl.BlockSpec(memory_space=pltpu.SMEM),
        ],
        out_shape=[
            jax.ShapeDtypeStruct((N_ROWS,), jnp.int32),
            jax.ShapeDtypeStruct((1,), jnp.int32),
        ],
        scratch_shapes=[pltpu.SMEM((1,), jnp.float32)],
    )(xt, a2, b)

    out = pl.pallas_call(
        _sub_body,
        grid=(GRID_B,),
        in_specs=[
            pl.BlockSpec((BS,), lambda i: (i,)),
            pl.BlockSpec(memory_space=pltpu.SMEM),
        ],
        out_specs=pl.BlockSpec((BS,), lambda i: (i,)),
        out_shape=jax.ShapeDtypeStruct((N_ROWS,), jnp.int32),
    )(hash_u, minv)
    return out
